# Initial kernel scaffold; baseline (speedup 1.0000x reference)
import jax
import jax.numpy as jnp
from jax.experimental import pallas as pl


def kernel(memory, last_update, src_nodes, dst_nodes, timestamps, event_features, indices, te_w, te_b):
    raise NotImplementedError("write your pallas kernel here")


# speedup vs baseline: 7.9918x; 7.9918x over previous
import functools
import math

import jax
import jax.numpy as jnp
from jax import lax
from jax.experimental import pallas as pl
from jax.experimental.pallas import tpu as pltpu
from jax.experimental.pallas import tpu_sc as plsc

N_NODES = 10000
N_EVENTS = 320000
D = 128

NC = 2
NS = 16
L = 16
NW = NC * NS
PER_TILE = N_EVENTS // NW
CH = 80
N_CHUNK = PER_TILE // CH

_COS_COEF = [(-1.0) ** k * (2.0 * math.pi) ** (2 * k) / math.factorial(2 * k)
             for k in range(8)]
_BIG = 1.5 * 2.0 ** 23


def _body(mem_hbm, lu_hbm, src_hbm, dst_hbm, ts_hbm, feat_hbm, idx_hbm,
          tw_hbm, tb_hbm, out_hbm,
          lu_v, tw_v, tb_v, sid_v, did_v, eid_v, ts_v, dt_c,
          bsrc, bdst, bfeat, btime, sem):
    wid = lax.axis_index("s") * NC + lax.axis_index("c")
    t0 = wid * PER_TILE

    pltpu.sync_copy(lu_hbm, lu_v)
    pltpu.sync_copy(tw_hbm, tw_v)
    pltpu.sync_copy(tb_hbm, tb_v)
    pltpu.sync_copy(src_hbm.at[pl.ds(t0, PER_TILE)], sid_v)
    pltpu.sync_copy(dst_hbm.at[pl.ds(t0, PER_TILE)], did_v)
    pltpu.sync_copy(idx_hbm.at[pl.ds(t0, PER_TILE)], eid_v)
    pltpu.sync_copy(ts_hbm.at[pl.ds(t0, PER_TILE)], ts_v)

    ws = tuple(tw_v[pl.ds(16 * j, 16)] for j in range(8))
    bs = tuple(tb_v[pl.ds(16 * j, 16)] for j in range(8))

    def chunk(c, carry):
        ws, bs = carry
        lo = c * CH
        base = t0 + lo

        d1 = pltpu.async_copy(mem_hbm.at[sid_v.at[pl.ds(lo, CH)]], bsrc, sem)
        d2 = pltpu.async_copy(mem_hbm.at[did_v.at[pl.ds(lo, CH)]], bdst, sem)
        d3 = pltpu.async_copy(feat_hbm.at[eid_v.at[pl.ds(lo, CH)]], bfeat, sem)

        for i in range(CH // L):
            idx16 = sid_v[pl.ds(lo + i * L, L)]
            lu16 = plsc.load_gather(lu_v, [idx16])
            dt_c[pl.ds(i * L, L)] = ts_v[pl.ds(lo + i * L, L)] - lu16

        def ev(e, carry):
            ws, bs = carry
            eidx = jnp.zeros((L,), jnp.int32) + e
            d = plsc.load_gather(dt_c, [eidx])
            for j in range(8):
                t = d * ws[j] + bs[j]
                n = (t + _BIG) - _BIG
                r = t - n
                s = r * r
                p = jnp.float32(_COS_COEF[7])
                for k in range(6, -1, -1):
                    p = p * s + jnp.float32(_COS_COEF[k])
                btime[e, pl.ds(16 * j, 16)] = p
            return carry
        lax.fori_loop(0, CH, ev, (ws, bs))

        d1.wait()
        d2.wait()
        d3.wait()

        pltpu.sync_copy(bsrc, out_hbm.at[pl.ds(base, CH), pl.ds(0, D)])
        pltpu.sync_copy(bdst, out_hbm.at[pl.ds(base, CH), pl.ds(D, D)])
        pltpu.sync_copy(btime, out_hbm.at[pl.ds(base, CH), pl.ds(2 * D, D)])
        pltpu.sync_copy(bfeat, out_hbm.at[pl.ds(base, CH), pl.ds(3 * D, D)])
        return (ws, bs)

    lax.fori_loop(0, N_CHUNK, chunk, (ws, bs))


_sc_call = pl.kernel(
    _body,
    out_type=jax.ShapeDtypeStruct((N_EVENTS, 4 * D), jnp.float32),
    mesh=plsc.VectorSubcoreMesh(core_axis_name="c", subcore_axis_name="s",
                                num_cores=NC, num_subcores=NS),
    compiler_params=pltpu.CompilerParams(needs_layout_passes=False),
    scratch_types=[
        pltpu.VMEM((N_NODES,), jnp.float32),
        pltpu.VMEM((D,), jnp.float32),
        pltpu.VMEM((D,), jnp.float32),
        pltpu.VMEM((PER_TILE,), jnp.int32),
        pltpu.VMEM((PER_TILE,), jnp.int32),
        pltpu.VMEM((PER_TILE,), jnp.int32),
        pltpu.VMEM((PER_TILE,), jnp.float32),
        pltpu.VMEM((CH,), jnp.float32),
        pltpu.VMEM((CH, D), jnp.float32),
        pltpu.VMEM((CH, D), jnp.float32),
        pltpu.VMEM((CH, D), jnp.float32),
        pltpu.VMEM((CH, D), jnp.float32),
        pltpu.SemaphoreType.DMA,
    ],
)


def kernel(memory, last_update, src_nodes, dst_nodes, timestamps,
           event_features, indices, te_w, te_b):
    inv2pi = jnp.float32(1.0 / (2.0 * math.pi))
    return _sc_call(
        memory,
        last_update,
        src_nodes.astype(jnp.int32),
        dst_nodes.astype(jnp.int32),
        timestamps,
        event_features,
        indices.astype(jnp.int32),
        (te_w * inv2pi).astype(jnp.float32),
        (te_b * inv2pi).astype(jnp.float32),
    )


# speedup vs baseline: 11.6825x; 1.4618x over previous
import math

import jax
import jax.numpy as jnp
from jax import lax
from jax.experimental import pallas as pl
from jax.experimental.pallas import tpu as pltpu
from jax.experimental.pallas import tpu_sc as plsc

N_NODES = 10000
N_EVENTS = 320000
D = 128

NC = 2
NS = 16
L = 16
NW = NC * NS
PER_TILE = N_EVENTS // NW
CH = 80
N_CHUNK = PER_TILE // CH

_COS_COEF = [(-1.0) ** k * (2.0 * math.pi) ** (2 * k) / math.factorial(2 * k)
             for k in range(8)]
_BIG = 1.5 * 2.0 ** 23


def _body(mem_hbm, lu_hbm, src_hbm, dst_hbm, ts_hbm, feat_hbm, idx_hbm,
          tw_hbm, tb_hbm, out_hbm,
          lu_v, tw_v, tb_v, sid_v, did_v, eid_v,
          ts_c, dt_c, bsrc, bdst, bfeat, btime,
          gsem, tssem0, tssem1, stsem0, stsem1):
    wid = lax.axis_index("s") * NC + lax.axis_index("c")
    t0 = wid * PER_TILE

    tssem = (tssem0, tssem1)
    stsem = (stsem0, stsem1)

    pltpu.sync_copy(lu_hbm, lu_v)
    pltpu.sync_copy(tw_hbm, tw_v)
    pltpu.sync_copy(tb_hbm, tb_v)
    pltpu.sync_copy(src_hbm.at[pl.ds(t0, PER_TILE)], sid_v)
    pltpu.sync_copy(dst_hbm.at[pl.ds(t0, PER_TILE)], did_v)
    pltpu.sync_copy(idx_hbm.at[pl.ds(t0, PER_TILE)], eid_v)

    ws = tuple(tw_v[pl.ds(16 * j, 16)] for j in range(8))
    bs = tuple(tb_v[pl.ds(16 * j, 16)] for j in range(8))

    pltpu.async_copy(ts_hbm.at[pl.ds(t0, CH)], ts_c.at[0], tssem[0])

    def process(c, b, drain, fire_ts):
        lo = c * CH
        base = t0 + lo
        nb = 1 - b

        if drain:
            for _ in range(4):
                pltpu.make_async_copy(
                    btime.at[b],
                    out_hbm.at[pl.ds(0, CH), pl.ds(0, D)],
                    stsem[b]).wait()

        d1 = pltpu.async_copy(mem_hbm.at[sid_v.at[pl.ds(lo, CH)]],
                              bsrc.at[b], gsem)
        d2 = pltpu.async_copy(mem_hbm.at[did_v.at[pl.ds(lo, CH)]],
                              bdst.at[b], gsem)
        d3 = pltpu.async_copy(feat_hbm.at[eid_v.at[pl.ds(lo, CH)]],
                              bfeat.at[b], gsem)
        if fire_ts:
            pltpu.async_copy(ts_hbm.at[pl.ds(base + CH, CH)],
                             ts_c.at[nb], tssem[nb])

        pltpu.make_async_copy(ts_hbm.at[pl.ds(0, CH)], ts_c.at[b],
                              tssem[b]).wait()
        for i in range(CH // L):
            idx16 = sid_v[pl.ds(lo + i * L, L)]
            lu16 = plsc.load_gather(lu_v, [idx16])
            dt_c[b, pl.ds(i * L, L)] = ts_c[b, pl.ds(i * L, L)] - lu16

        def ev(e, carry):
            eidx = jnp.zeros((L,), jnp.int32) + e
            d = plsc.load_gather(dt_c.at[b], [eidx])
            for j in range(8):
                t = d * ws[j] + bs[j]
                n = (t + _BIG) - _BIG
                r = t - n
                s = r * r
                p = jnp.float32(_COS_COEF[7])
                for k in range(6, -1, -1):
                    p = p * s + jnp.float32(_COS_COEF[k])
                btime[b, e, pl.ds(16 * j, 16)] = p
            return carry
        lax.fori_loop(0, CH, ev, 0)

        d1.wait()
        d2.wait()
        d3.wait()

        pltpu.async_copy(bsrc.at[b],
                         out_hbm.at[pl.ds(base, CH), pl.ds(0, D)], stsem[b])
        pltpu.async_copy(bdst.at[b],
                         out_hbm.at[pl.ds(base, CH), pl.ds(D, D)], stsem[b])
        pltpu.async_copy(btime.at[b],
                         out_hbm.at[pl.ds(base, CH), pl.ds(2 * D, D)],
                         stsem[b])
        pltpu.async_copy(bfeat.at[b],
                         out_hbm.at[pl.ds(base, CH), pl.ds(3 * D, D)],
                         stsem[b])

    def pair(i, carry):
        for b in range(2):
            @pl.when(i >= 1)
            def _(b=b):
                for _ in range(4):
                    pltpu.make_async_copy(
                        btime.at[b],
                        out_hbm.at[pl.ds(0, CH), pl.ds(0, D)],
                        stsem[b]).wait()
            process(2 * i + b, b, drain=False, fire_ts=True)
        return carry

    lax.fori_loop(0, N_CHUNK // 2, pair, 0)
    process(jnp.int32(N_CHUNK - 1), 0, drain=True, fire_ts=False)

    for b in range(2):
        for _ in range(4):
            pltpu.make_async_copy(
                btime.at[b],
                out_hbm.at[pl.ds(0, CH), pl.ds(0, D)],
                stsem[b]).wait()


_sc_call = pl.kernel(
    _body,
    out_type=jax.ShapeDtypeStruct((N_EVENTS, 4 * D), jnp.float32),
    mesh=plsc.VectorSubcoreMesh(core_axis_name="c", subcore_axis_name="s",
                                num_cores=NC, num_subcores=NS),
    compiler_params=pltpu.CompilerParams(needs_layout_passes=False),
    scratch_types=[
        pltpu.VMEM((N_NODES,), jnp.float32),
        pltpu.VMEM((D,), jnp.float32),
        pltpu.VMEM((D,), jnp.float32),
        pltpu.VMEM((PER_TILE,), jnp.int32),
        pltpu.VMEM((PER_TILE,), jnp.int32),
        pltpu.VMEM((PER_TILE,), jnp.int32),
        pltpu.VMEM((2, CH), jnp.float32),
        pltpu.VMEM((2, CH), jnp.float32),
        pltpu.VMEM((2, CH, D), jnp.float32),
        pltpu.VMEM((2, CH, D), jnp.float32),
        pltpu.VMEM((2, CH, D), jnp.float32),
        pltpu.VMEM((2, CH, D), jnp.float32),
        pltpu.SemaphoreType.DMA,
        pltpu.SemaphoreType.DMA,
        pltpu.SemaphoreType.DMA,
        pltpu.SemaphoreType.DMA,
        pltpu.SemaphoreType.DMA,
    ],
)


def kernel(memory, last_update, src_nodes, dst_nodes, timestamps,
           event_features, indices, te_w, te_b):
    inv2pi = jnp.float32(1.0 / (2.0 * math.pi))
    return _sc_call(
        memory,
        last_update,
        src_nodes.astype(jnp.int32),
        dst_nodes.astype(jnp.int32),
        timestamps,
        event_features,
        indices.astype(jnp.int32),
        (te_w * inv2pi).astype(jnp.float32),
        (te_b * inv2pi).astype(jnp.float32),
    )
